# baseline (speedup 1.0000x reference)
import jax
import jax.numpy as jnp
from jax.experimental import pallas as pl
from jax.experimental.pallas import tpu as pltpu

_BF = jnp.bfloat16
_F32 = jnp.float32


def _softmax_rows(s):
    s = s - jnp.max(s, axis=-1, keepdims=True)
    e = jnp.exp(s)
    return e / jnp.sum(e, axis=-1, keepdims=True)


def _attn_kernel(x_ref, wqkv_ref, wog_ref, bt_ref, o_ref):
    BB, T, D = x_ref.shape
    G = o_ref.shape[-1]

    xb = x_ref[...].reshape(BB * T, D).astype(_BF)
    qkv = jnp.dot(xb, wqkv_ref[...], preferred_element_type=_F32)

    dn = (((1,), (1,)), ((), ()))
    outs = []
    for i in range(BB):
        rows = qkv[i * T:(i + 1) * T]
        q_t = rows[:, 0 * D:1 * D]
        k_t = rows[:, 1 * D:2 * D]
        v_t = rows[:, 2 * D:3 * D]
        q_d = rows[:, 3 * D:4 * D]
        k_d = rows[:, 4 * D:5 * D]
        v_d = rows[:, 5 * D:6 * D]
        p_t = _softmax_rows(
            jax.lax.dot_general(q_d, k_t, dn, preferred_element_type=_F32))
        p_d = _softmax_rows(
            jax.lax.dot_general(q_t, k_d, dn, preferred_element_type=_F32))
        o_t = jnp.dot(p_t, v_t, preferred_element_type=_F32)
        o_d = jnp.dot(p_d, v_d, preferred_element_type=_F32)
        outs.append(jnp.concatenate([o_t, o_d], axis=-1))

    cat = jnp.concatenate(outs, axis=0).astype(_BF)
    og = jnp.dot(cat, wog_ref[...], preferred_element_type=_F32) + bt_ref[...]
    o_ref[...] = og.reshape(BB, T, G)


def _rec_kernel(gx_ref, wa_ref, wbd_ref, wc_ref, bd_ref,
                wtf_ref, btf_ref, wdf_ref, bdf_ref,
                out_t_ref, out_d_ref,
                h_t_s, c_t_s, h_d_s, c_d_s, hs_t, hs_d):
    TS, B, G = gx_ref.shape
    H = G // 4

    @pl.when(pl.program_id(0) == 0)
    def _init():
        h_t_s[...] = jnp.zeros_like(h_t_s)
        c_t_s[...] = jnp.zeros_like(c_t_s)
        h_d_s[...] = jnp.zeros_like(h_d_s)
        c_d_s[...] = jnp.zeros_like(c_d_s)

    wa = wa_ref[...]
    wbd = wbd_ref[...]
    wc = wc_ref[...]
    bd = bd_ref[...]

    def cell(gates, c_prev):
        i = jax.nn.sigmoid(gates[:, 0 * H:1 * H])
        f = jax.nn.sigmoid(gates[:, 1 * H:2 * H])
        g = jnp.tanh(gates[:, 2 * H:3 * H])
        o = jax.nn.sigmoid(gates[:, 3 * H:4 * H])
        c_new = f * c_prev + i * g
        return o * jnp.tanh(c_new), c_new

    h_t = h_t_s[...]
    c_t = c_t_s[...]
    h_d = h_d_s[...]
    c_d = c_d_s[...]

    for ts in range(TS):
        m1 = jnp.dot(h_d.astype(_BF), wbd, preferred_element_type=_F32)
        m2 = jnp.dot(h_t.astype(_BF), wa, preferred_element_type=_F32)
        gates_t = gx_ref[ts] + m2 + m1[:, :G]
        h_t, c_t = cell(gates_t, c_t)
        m3 = jnp.dot(h_t.astype(_BF), wc, preferred_element_type=_F32)
        h_d, c_d = cell(m3 + m1[:, G:] + bd, c_d)
        hs_t[pl.ds(ts * B, B), :] = h_t
        hs_d[pl.ds(ts * B, B), :] = h_d

    h_t_s[...] = h_t
    c_t_s[...] = c_t
    h_d_s[...] = h_d
    c_d_s[...] = c_d

    out_t_ref[...] = (jnp.dot(hs_t[...].astype(_BF), wtf_ref[...],
                              preferred_element_type=_F32) + btf_ref[...])
    out_d_ref[...] = (jnp.dot(hs_d[...].astype(_BF), wdf_ref[...],
                              preferred_element_type=_F32) + bdf_ref[...])


def kernel(x, w_qkv, w_out_gates, w_gates_t, b_gates_t, w_gates_d, b_gates_d,
           net_t_w, net_t_b, net_d_w, net_d_b):
    T, B, D = x.shape
    H = net_t_w.shape[0]
    G = 4 * H
    BB = max(1, B // 2)

    x_btd = jnp.transpose(x, (1, 0, 2))
    gx_btg = pl.pallas_call(
        _attn_kernel,
        out_shape=jax.ShapeDtypeStruct((B, T, G), _F32),
        grid=(B // BB,),
        in_specs=[pl.BlockSpec((BB, T, D), lambda i: (i, 0, 0)),
                  pl.BlockSpec((D, 6 * D), lambda i: (0, 0)),
                  pl.BlockSpec((2 * D, G), lambda i: (0, 0)),
                  pl.BlockSpec((1, G), lambda i: (0, 0))],
        out_specs=pl.BlockSpec((BB, T, G), lambda i: (i, 0, 0)),
        compiler_params=pltpu.CompilerParams(
            dimension_semantics=("parallel",)),
    )(x_btd, w_qkv.astype(_BF), w_out_gates.astype(_BF), b_gates_t)
    gx_tbg = jnp.transpose(gx_btg, (1, 0, 2))

    TS = 8
    while T % TS:
        TS -= 1
    wa = w_gates_t[:H].astype(_BF)
    wbd = jnp.concatenate([w_gates_t[H:], w_gates_d[H:]], axis=1).astype(_BF)
    wc = w_gates_d[:H].astype(_BF)

    def full(shape):
        return pl.BlockSpec(shape, lambda g: (0,) * len(shape))

    out_t, out_d = pl.pallas_call(
        _rec_kernel,
        out_shape=(jax.ShapeDtypeStruct((T * B, D), _F32),
                   jax.ShapeDtypeStruct((T * B, D), _F32)),
        grid=(T // TS,),
        in_specs=[pl.BlockSpec((TS, B, G), lambda g: (g, 0, 0)),
                  full((H, G)), full((H, 2 * G)), full((H, G)), full((1, G)),
                  full((H, D)), full((1, D)), full((H, D)), full((1, D))],
        out_specs=[pl.BlockSpec((TS * B, D), lambda g: (g, 0)),
                   pl.BlockSpec((TS * B, D), lambda g: (g, 0))],
        scratch_shapes=[pltpu.VMEM((B, H), _F32)] * 4
                     + [pltpu.VMEM((TS * B, H), _F32)] * 2,
        compiler_params=pltpu.CompilerParams(
            dimension_semantics=("arbitrary",)),
    )(gx_tbg, wa, wbd, wc, b_gates_d,
      net_t_w.astype(_BF), net_t_b, net_d_w.astype(_BF), net_d_b)
    return out_t.reshape(T, B, D), out_d.reshape(T, B, D)
